# baseline (speedup 1.0000x reference)
import functools

import jax
import jax.numpy as jnp
from jax.experimental import pallas as pl
from jax.experimental.pallas import tpu as pltpu


def _round_up(x, m):
    return (x + m - 1) // m * m


def _mm_kernel(x_ref, w_ref, o_ref, *, nk, tk, k_tail):
    k = pl.program_id(0)
    bb, s, hp = o_ref.shape

    def partial_dot(masked):
        x = x_ref[...]
        w = w_ref[...]
        if masked:
            xcol = jax.lax.broadcasted_iota(jnp.int32, x.shape, 1)
            x = jnp.where(xcol < k_tail, x, 0.0)
        out = jnp.dot(
            x.astype(jnp.bfloat16),
            w.astype(jnp.bfloat16),
            preferred_element_type=jnp.float32,
        )
        return out.reshape(bb, s, hp)

    @pl.when(k == 0)
    def _():
        o_ref[...] = partial_dot(masked=(nk == 1 and k_tail != tk))

    @pl.when(jnp.logical_and(k > 0, k < nk - 1))
    def _():
        o_ref[...] += partial_dot(masked=False)

    if nk > 1:
        @pl.when(k == nk - 1)
        def _():
            o_ref[...] += partial_dot(masked=(k_tail != tk))


def kernel(weight, mask):
    B, S, V = mask.shape
    Vw, H = weight.shape
    M = B * S

    tk = 2048
    nk = -(-V // tk)
    k_tail = V - (nk - 1) * tk

    x = mask.reshape(M, V)
    Hp = _round_up(H, 128)
    Vp = nk * tk
    w = jnp.pad(weight, ((0, Vp - V), (0, Hp - H)))

    out = pl.pallas_call(
        functools.partial(_mm_kernel, nk=nk, tk=tk, k_tail=k_tail),
        out_shape=jax.ShapeDtypeStruct((B, S, Hp), weight.dtype),
        grid=(nk,),
        in_specs=[
            pl.BlockSpec((M, tk), lambda k: (0, k)),
            pl.BlockSpec((tk, Hp), lambda k: (k, 0)),
        ],
        out_specs=pl.BlockSpec((B, S, Hp), lambda k: (0, 0, 0)),
        compiler_params=pltpu.CompilerParams(
            dimension_semantics=("arbitrary",),
            vmem_limit_bytes=100 * 1024 * 1024,
        ),
    )(x, w)
    if Hp != H:
        out = out[..., :H]
    return out


# speedup vs baseline: 1.1681x; 1.1681x over previous
import functools

import jax
import jax.numpy as jnp
from jax.experimental import pallas as pl
from jax.experimental.pallas import tpu as pltpu


def _round_up(x, m):
    return (x + m - 1) // m * m


def _mm_kernel(x_ref, w_ref, o_ref, *, nk, tk, k_tail):
    k = pl.program_id(0)

    def partial_dot(masked):
        x = x_ref[...]
        w = w_ref[...]
        if masked:
            xcol = jax.lax.broadcasted_iota(jnp.int32, x.shape, 1)
            wrow = jax.lax.broadcasted_iota(jnp.int32, w.shape, 0)
            x = jnp.where(xcol < k_tail, x, 0.0)
            w = jnp.where(wrow < k_tail, w, 0.0)
        out = jnp.dot(
            x.astype(jnp.bfloat16),
            w.astype(jnp.bfloat16),
            preferred_element_type=jnp.float32,
        )
        return out.reshape(o_ref.shape)

    @pl.when(k == 0)
    def _():
        o_ref[...] = partial_dot(masked=(nk == 1 and k_tail != tk))

    @pl.when(jnp.logical_and(k > 0, k < nk - 1))
    def _():
        o_ref[...] += partial_dot(masked=False)

    if nk > 1:
        @pl.when(k == nk - 1)
        def _():
            o_ref[...] += partial_dot(masked=(k_tail != tk))


def kernel(weight, mask):
    B, S, V = mask.shape
    Vw, H = weight.shape
    M = B * S
    x = mask.reshape(M, V)

    Hp = _round_up(H, 128)
    w = weight if Hp == H else jnp.pad(weight, ((0, 0), (0, Hp - H)))

    tk = 2048
    nk = -(-V // tk)
    k_tail = V - (nk - 1) * tk

    out = pl.pallas_call(
        functools.partial(_mm_kernel, nk=nk, tk=tk, k_tail=k_tail),
        out_shape=jax.ShapeDtypeStruct((B, S, Hp), weight.dtype),
        grid=(nk,),
        in_specs=[
            pl.BlockSpec((M, tk), lambda k: (0, k)),
            pl.BlockSpec((tk, Hp), lambda k: (k, 0)),
        ],
        out_specs=pl.BlockSpec((B, S, Hp), lambda k: (0, 0, 0)),
        compiler_params=pltpu.CompilerParams(
            dimension_semantics=("arbitrary",),
            vmem_limit_bytes=100 * 1024 * 1024,
        ),
    )(x, w)
    return out[..., :H] if Hp != H else out
